# baseline (speedup 1.0000x reference)
import functools

import jax
import jax.numpy as jnp
from jax import lax
from jax.experimental import pallas as pl
from jax.experimental.pallas import tpu as pltpu
from jax.experimental.pallas import tpu_sc as plsc

N = 10000
E = 320000
D = 128
H = 128
NC = 2
NS = 16
NP = 10240
HH = 64

CHUNK_A = 128
CHUNK_B = 128

ET_A = 10240
NCH_A = ET_A // CHUNK_A
EP_A = ET_A * NC * NS

ET_B = 20480
NCH_B = ET_B // CHUNK_B
EP_B = ET_B * NS

_MESH = dict(core_axis_name="c", subcore_axis_name="s", num_cores=NC,
             num_subcores=NS)


def _sync_agg(tbl_hbm, src_v, dst_v, rows_v, acc, sem, nch):
    def step(j, carry):
        pltpu.async_copy(tbl_hbm.at[src_v.at[j]], rows_v, sem).wait()
        pltpu.sync_copy(rows_v, acc.at[dst_v.at[j]], add=True)
        return carry

    lax.fori_loop(0, nch, step, 0, unroll=False)


def _sc_agg16_body(tbl_hbm, srcr_hbm, dstr_hbm, zer_hbm, out_hbm,
                   src_v, dst_v, rows_v, acc, semg):
    cid = lax.axis_index("c")
    sid = lax.axis_index("s")
    rpt = NP // NS
    r0 = sid * rpt
    pltpu.sync_copy(zer_hbm.at[pl.ds(r0, rpt)], acc.at[pl.ds(r0, rpt)])
    pltpu.sync_copy(srcr_hbm.at[cid, sid], src_v)
    pltpu.sync_copy(dstr_hbm.at[cid, sid], dst_v)
    plsc.subcore_barrier()
    _sync_agg(tbl_hbm, src_v, dst_v, rows_v, acc, semg, NCH_A)
    plsc.subcore_barrier()
    pltpu.sync_copy(acc.at[pl.ds(r0, rpt)], out_hbm.at[cid, pl.ds(r0, rpt)])


@functools.lru_cache(maxsize=None)
def _sc_agg16():
    return pl.kernel(
        _sc_agg16_body,
        out_type=jax.ShapeDtypeStruct((NC, NP, 16), jnp.float32),
        mesh=plsc.VectorSubcoreMesh(**_MESH),
        scratch_types=[
            pltpu.VMEM((NCH_A, CHUNK_A), jnp.int32),
            pltpu.VMEM((NCH_A, CHUNK_A), jnp.int32),
            pltpu.VMEM((CHUNK_A, 16), jnp.float32),
            pltpu.VMEM_SHARED((NP, 16), jnp.float32),
            pltpu.SemaphoreType.DMA,
        ],
        compiler_params=pltpu.CompilerParams(use_tc_tiling_on_sc=False),
    )


def _sc_agg_big_body(hsx_hbm, src2_hbm, dstp_hbm, zer_hbm, out_hbm,
                     src_v, dst_v, rows_v, acc, semg):
    cid = lax.axis_index("c")
    sid = lax.axis_index("s")
    rpt = NP // NS
    r0 = sid * rpt
    pltpu.sync_copy(zer_hbm.at[pl.ds(r0, rpt)], acc.at[pl.ds(r0, rpt)])
    pltpu.sync_copy(src2_hbm.at[cid, sid], src_v)
    pltpu.sync_copy(dstp_hbm.at[sid], dst_v)
    plsc.subcore_barrier()
    _sync_agg(hsx_hbm, src_v, dst_v, rows_v, acc, semg, NCH_B)
    plsc.subcore_barrier()
    pltpu.sync_copy(acc.at[pl.ds(r0, rpt)],
                    out_hbm.at[pl.ds(r0, rpt), pl.ds(cid * HH, HH)])


@functools.lru_cache(maxsize=None)
def _sc_agg_big():
    return pl.kernel(
        _sc_agg_big_body,
        out_type=jax.ShapeDtypeStruct((NP, 2 * HH), jnp.float32),
        mesh=plsc.VectorSubcoreMesh(**_MESH),
        scratch_types=[
            pltpu.VMEM((NCH_B, CHUNK_B), jnp.int32),
            pltpu.VMEM((NCH_B, CHUNK_B), jnp.int32),
            pltpu.VMEM((CHUNK_B, HH), jnp.float32),
            pltpu.VMEM_SHARED((NP, HH), jnp.float32),
            pltpu.SemaphoreType.DMA,
        ],
        compiler_params=pltpu.CompilerParams(use_tc_tiling_on_sc=False),
    )


def _tc1_body(degp, x, w1, sp, dis_o, hs1_o, src2_o):
    d = degp[...]
    deg = d[0][:N, 0:1] + d[1][:N, 0:1] + 1.0
    dis = 1.0 / jnp.sqrt(deg)
    dis_o[...] = dis
    h = jnp.dot(x[...], w1[...], preferred_element_type=jnp.float32)
    hs1_o[...] = h * dis
    srcm = sp[...]
    src2_o[0] = srcm * 2
    src2_o[1] = srcm * 2 + 1


_tc1 = pl.pallas_call(
    _tc1_body,
    out_shape=[
        jax.ShapeDtypeStruct((N, 1), jnp.float32),
        jax.ShapeDtypeStruct((N, H), jnp.float32),
        jax.ShapeDtypeStruct((NC, EP_B // 128, 128), jnp.int32),
    ],
)


def _tc_mid_body(agg, hs, dis, b, g, bt, wn, out, *, pad16):
    t = (agg[:N] + hs[...]) * dis[...] + b[...][None, :]
    mu = jnp.mean(t, axis=0, keepdims=True)
    tc = t - mu
    var = jnp.mean(tc * tc, axis=0, keepdims=True)
    y = g[...][None, :] * tc / jnp.sqrt(var + 1e-5) + bt[...][None, :]
    z = jnp.where(y >= 0, y, 0.01 * y)
    hn = jnp.dot(z, wn[...], preferred_element_type=jnp.float32) * dis[...]
    if pad16:
        col = lax.broadcasted_iota(jnp.int32, (1, 16), 1)
        out[:N] = jnp.where(col == 0, hn, 0.0)
        out[N:] = jnp.zeros((NP - N, 16), jnp.float32)
    else:
        out[...] = hn


_tc_mid128 = pl.pallas_call(
    functools.partial(_tc_mid_body, pad16=False),
    out_shape=jax.ShapeDtypeStruct((N, H), jnp.float32),
)

_tc_mid16 = pl.pallas_call(
    functools.partial(_tc_mid_body, pad16=True),
    out_shape=jax.ShapeDtypeStruct((NP, 16), jnp.float32),
)


def _tc3_body(aggp, hs3, dis, b3, out):
    a = aggp[...]
    s = a[0][:N, 0:1] + a[1][:N, 0:1] + hs3[:N, 0:1]
    out[...] = s * dis[...] + b3[...]


_tc3 = pl.pallas_call(
    _tc3_body,
    out_shape=jax.ShapeDtypeStruct((N, 1), jnp.float32),
)


def _pad_edges(v, per_real, per_pad, pad_val, shape):
    r = v.reshape(-1, per_real)
    p = jnp.full((r.shape[0], per_pad - per_real), pad_val, jnp.int32)
    return jnp.concatenate([r, p], axis=1).reshape(shape)


def kernel(x, edge_index, W1, b1, g1, bt1, W2, b2, g2, bt2, W3, b3):
    src = edge_index[0]
    dst = edge_index[1]

    padn = EP_A - E
    src_a = jnp.concatenate([src, jnp.zeros((padn,), jnp.int32)])
    dst_a = jnp.concatenate([dst, jnp.full((padn,), N, jnp.int32)])
    src_r16 = src_a.reshape(NC, NS, NCH_A, CHUNK_A)
    dst_r16 = dst_a.reshape(NC, NS, NCH_A, CHUNK_A)

    src_bp = _pad_edges(src, E // NS, ET_B, 0, (EP_B // 128, 128))
    dst_bp = _pad_edges(dst, E // NS, ET_B, N, (NS, NCH_B, CHUNK_B))

    ones16 = jnp.ones((NP, 16), jnp.float32)
    zeros16 = jnp.zeros((NP, 16), jnp.float32)
    zeros64 = jnp.zeros((NP, HH), jnp.float32)

    degp = _sc_agg16()(ones16, dst_r16, dst_r16, zeros16)

    dis, hs1, src2m = _tc1(degp, x, W1, src_bp)
    src2 = src2m.reshape(NC, NS, NCH_B, CHUNK_B)

    agg1 = _sc_agg_big()(hs1.reshape(2 * N, HH), src2, dst_bp, zeros64)
    hs2 = _tc_mid128(agg1, hs1, dis, b1, g1, bt1, W2)

    agg2 = _sc_agg_big()(hs2.reshape(2 * N, HH), src2, dst_bp, zeros64)
    hs3p = _tc_mid16(agg2, hs2, dis, b2, g2, bt2, W3)

    agg3 = _sc_agg16()(hs3p, src_r16, dst_r16, zeros16)
    return _tc3(agg3, hs3p, dis, b3)


# speedup vs baseline: 1.5437x; 1.5437x over previous
import functools

import jax
import jax.numpy as jnp
from jax import lax
from jax.experimental import pallas as pl
from jax.experimental.pallas import tpu as pltpu
from jax.experimental.pallas import tpu_sc as plsc

N = 10000
E = 320000
D = 128
H = 128
NC = 2
NS = 16
NP = 10240
HH = 64

CHUNK_A = 128
CHUNK_B = 80

ET_A = 10240
NCH_A = ET_A // CHUNK_A
EP_A = ET_A * NC * NS

ET_B = E // NS
NCH_B = ET_B // CHUNK_B
EP_B = E

_MESH = dict(core_axis_name="c", subcore_axis_name="s", num_cores=NC,
             num_subcores=NS)


def _sync_agg(tbl_hbm, src_v, dst_v, rows_v, acc, sem, nch):
    def step(j, carry):
        pltpu.async_copy(tbl_hbm.at[src_v.at[j]], rows_v, sem).wait()
        pltpu.sync_copy(rows_v, acc.at[dst_v.at[j]], add=True)
        return carry

    lax.fori_loop(0, nch, step, 0, unroll=False)


def _sc_agg16_body(tbl_hbm, srcr_hbm, dstr_hbm, zer_hbm, out_hbm,
                   src_v, dst_v, rows_v, acc, semg):
    cid = lax.axis_index("c")
    sid = lax.axis_index("s")
    rpt = NP // NS
    r0 = sid * rpt
    pltpu.sync_copy(zer_hbm.at[pl.ds(r0, rpt)], acc.at[pl.ds(r0, rpt)])
    pltpu.sync_copy(srcr_hbm.at[cid, sid], src_v)
    pltpu.sync_copy(dstr_hbm.at[cid, sid], dst_v)
    plsc.subcore_barrier()
    _sync_agg(tbl_hbm, src_v, dst_v, rows_v, acc, semg, NCH_A)
    plsc.subcore_barrier()
    pltpu.sync_copy(acc.at[pl.ds(r0, rpt)], out_hbm.at[cid, pl.ds(r0, rpt)])


@functools.lru_cache(maxsize=None)
def _sc_agg16():
    return pl.kernel(
        _sc_agg16_body,
        out_type=jax.ShapeDtypeStruct((NC, NP, 16), jnp.float32),
        mesh=plsc.VectorSubcoreMesh(**_MESH),
        scratch_types=[
            pltpu.VMEM((NCH_A, CHUNK_A), jnp.int32),
            pltpu.VMEM((NCH_A, CHUNK_A), jnp.int32),
            pltpu.VMEM((CHUNK_A, 16), jnp.float32),
            pltpu.VMEM_SHARED((NP, 16), jnp.float32),
            pltpu.SemaphoreType.DMA,
        ],
        compiler_params=pltpu.CompilerParams(use_tc_tiling_on_sc=False),
    )


def _sc_agg_big_body(hsx_hbm, src2_hbm, dstp_hbm, zer_hbm, out_hbm,
                     src_v, dst_v, rows_v, acc, semg):
    cid = lax.axis_index("c")
    sid = lax.axis_index("s")
    rpt = NP // NS
    r0 = sid * rpt
    pltpu.sync_copy(zer_hbm.at[pl.ds(r0, rpt)], acc.at[pl.ds(r0, rpt)])
    pltpu.sync_copy(src2_hbm.at[cid, sid], src_v)
    pltpu.sync_copy(dstp_hbm.at[sid], dst_v)
    plsc.subcore_barrier()
    _sync_agg(hsx_hbm, src_v, dst_v, rows_v, acc, semg, NCH_B)
    plsc.subcore_barrier()
    pltpu.sync_copy(acc.at[pl.ds(r0, rpt)],
                    out_hbm.at[pl.ds(r0, rpt), pl.ds(cid * HH, HH)])


@functools.lru_cache(maxsize=None)
def _sc_agg_big():
    return pl.kernel(
        _sc_agg_big_body,
        out_type=jax.ShapeDtypeStruct((NP, 2 * HH), jnp.float32),
        mesh=plsc.VectorSubcoreMesh(**_MESH),
        scratch_types=[
            pltpu.VMEM((NCH_B, CHUNK_B), jnp.int32),
            pltpu.VMEM((NCH_B, CHUNK_B), jnp.int32),
            pltpu.VMEM((CHUNK_B, HH), jnp.float32),
            pltpu.VMEM_SHARED((NP, HH), jnp.float32),
            pltpu.SemaphoreType.DMA,
        ],
        compiler_params=pltpu.CompilerParams(use_tc_tiling_on_sc=False),
    )


def _tc1_body(degp, x, w1, sp, dis_o, hs1_o, src2_o):
    d = degp[...]
    deg = d[0][:N, 0:1] + d[1][:N, 0:1] + 1.0
    dis = 1.0 / jnp.sqrt(deg)
    dis_o[...] = dis
    h = jnp.dot(x[...], w1[...], preferred_element_type=jnp.float32)
    hs1_o[...] = h * dis
    srcm = sp[...]
    src2_o[0] = srcm * 2
    src2_o[1] = srcm * 2 + 1


_tc1 = pl.pallas_call(
    _tc1_body,
    out_shape=[
        jax.ShapeDtypeStruct((N, 1), jnp.float32),
        jax.ShapeDtypeStruct((N, H), jnp.float32),
        jax.ShapeDtypeStruct((NC, EP_B // 128, 128), jnp.int32),
    ],
)


def _tc_mid_body(agg, hs, dis, b, g, bt, wn, out, *, pad16):
    t = (agg[:N] + hs[...]) * dis[...] + b[...][None, :]
    mu = jnp.mean(t, axis=0, keepdims=True)
    tc = t - mu
    var = jnp.mean(tc * tc, axis=0, keepdims=True)
    y = g[...][None, :] * tc / jnp.sqrt(var + 1e-5) + bt[...][None, :]
    z = jnp.where(y >= 0, y, 0.01 * y)
    hn = jnp.dot(z, wn[...], preferred_element_type=jnp.float32) * dis[...]
    if pad16:
        col = lax.broadcasted_iota(jnp.int32, (1, 16), 1)
        out[:N] = jnp.where(col == 0, hn, 0.0)
        out[N:] = jnp.zeros((NP - N, 16), jnp.float32)
    else:
        out[...] = hn


_tc_mid128 = pl.pallas_call(
    functools.partial(_tc_mid_body, pad16=False),
    out_shape=jax.ShapeDtypeStruct((N, H), jnp.float32),
)

_tc_mid16 = pl.pallas_call(
    functools.partial(_tc_mid_body, pad16=True),
    out_shape=jax.ShapeDtypeStruct((NP, 16), jnp.float32),
)


def _tc3_body(aggp, hs3, dis, b3, out):
    a = aggp[...]
    s = a[0][:N, 0:1] + a[1][:N, 0:1] + hs3[:N, 0:1]
    out[...] = s * dis[...] + b3[...]


_tc3 = pl.pallas_call(
    _tc3_body,
    out_shape=jax.ShapeDtypeStruct((N, 1), jnp.float32),
)


def _pad_edges(v, per_real, per_pad, pad_val, shape):
    r = v.reshape(-1, per_real)
    p = jnp.full((r.shape[0], per_pad - per_real), pad_val, jnp.int32)
    return jnp.concatenate([r, p], axis=1).reshape(shape)


def kernel(x, edge_index, W1, b1, g1, bt1, W2, b2, g2, bt2, W3, b3):
    src = edge_index[0]
    dst = edge_index[1]

    padn = EP_A - E
    src_a = jnp.concatenate([src, jnp.zeros((padn,), jnp.int32)])
    dst_a = jnp.concatenate([dst, jnp.full((padn,), N, jnp.int32)])
    src_r16 = src_a.reshape(NC, NS, NCH_A, CHUNK_A)
    dst_r16 = dst_a.reshape(NC, NS, NCH_A, CHUNK_A)

    src_bp = src.reshape(EP_B // 128, 128)
    dst_bp = dst.reshape(NS, NCH_B, CHUNK_B)

    ones16 = jnp.ones((NP, 16), jnp.float32)
    zeros16 = jnp.zeros((NP, 16), jnp.float32)
    zeros64 = jnp.zeros((NP, HH), jnp.float32)

    degp = _sc_agg16()(ones16, dst_r16, dst_r16, zeros16)

    dis, hs1, src2m = _tc1(degp, x, W1, src_bp)
    src2 = src2m.reshape(NC, NS, NCH_B, CHUNK_B)

    agg1 = _sc_agg_big()(hs1.reshape(2 * N, HH), src2, dst_bp, zeros64)
    hs2 = _tc_mid128(agg1, hs1, dis, b1, g1, bt1, W2)

    agg2 = _sc_agg_big()(hs2.reshape(2 * N, HH), src2, dst_bp, zeros64)
    hs3p = _tc_mid16(agg2, hs2, dis, b2, g2, bt2, W3)

    agg3 = _sc_agg16()(hs3p, src_r16, dst_r16, zeros16)
    return _tc3(agg3, hs3p, dis, b3)


# speedup vs baseline: 1.6109x; 1.0435x over previous
import functools

import jax
import jax.numpy as jnp
from jax import lax
from jax.experimental import pallas as pl
from jax.experimental.pallas import tpu as pltpu
from jax.experimental.pallas import tpu_sc as plsc

N = 10000
E = 320000
D = 128
H = 128
NC = 2
NS = 16
NP = 10240
HH = 64

CHUNK_A = 256
CHUNK_B = 80

ET_A = 10240
NCH_A = ET_A // CHUNK_A
EP_A = ET_A * NC * NS

ET_B = E // NS
NCH_B = ET_B // CHUNK_B
EP_B = E

_MESH = dict(core_axis_name="c", subcore_axis_name="s", num_cores=NC,
             num_subcores=NS)


def _sync_agg(tbl_hbm, src_v, dst_v, rows_v, acc, sem, nch):
    def step(j, carry):
        pltpu.async_copy(tbl_hbm.at[src_v.at[j]], rows_v, sem).wait()
        pltpu.sync_copy(rows_v, acc.at[dst_v.at[j]], add=True)
        return carry

    lax.fori_loop(0, nch, step, 0, unroll=False)


def _sc_agg16_body(tbl_hbm, srcr_hbm, dstr_hbm, zer_hbm, out_hbm,
                   src_v, dst_v, rows_v, acc, semg):
    cid = lax.axis_index("c")
    sid = lax.axis_index("s")
    rpt = NP // NS
    r0 = sid * rpt
    pltpu.sync_copy(zer_hbm.at[pl.ds(r0, rpt)], acc.at[pl.ds(r0, rpt)])
    pltpu.sync_copy(srcr_hbm.at[cid, sid], src_v)
    pltpu.sync_copy(dstr_hbm.at[cid, sid], dst_v)
    plsc.subcore_barrier()
    _sync_agg(tbl_hbm, src_v, dst_v, rows_v, acc, semg, NCH_A)
    plsc.subcore_barrier()
    pltpu.sync_copy(acc.at[pl.ds(r0, rpt)], out_hbm.at[cid, pl.ds(r0, rpt)])


@functools.lru_cache(maxsize=None)
def _sc_agg16():
    return pl.kernel(
        _sc_agg16_body,
        out_type=jax.ShapeDtypeStruct((NC, NP, 16), jnp.float32),
        mesh=plsc.VectorSubcoreMesh(**_MESH),
        scratch_types=[
            pltpu.VMEM((NCH_A, CHUNK_A), jnp.int32),
            pltpu.VMEM((NCH_A, CHUNK_A), jnp.int32),
            pltpu.VMEM((CHUNK_A, 16), jnp.float32),
            pltpu.VMEM_SHARED((NP, 16), jnp.float32),
            pltpu.SemaphoreType.DMA,
        ],
        compiler_params=pltpu.CompilerParams(use_tc_tiling_on_sc=False),
    )


def _sc_agg_big_body(hsx_hbm, src2_hbm, dstp_hbm, zer_hbm, out_hbm,
                     src_v, dst_v, rows_v, acc, semg):
    cid = lax.axis_index("c")
    sid = lax.axis_index("s")
    rpt = NP // NS
    r0 = sid * rpt
    pltpu.sync_copy(zer_hbm.at[pl.ds(r0, rpt)], acc.at[pl.ds(r0, rpt)])
    pltpu.sync_copy(src2_hbm.at[cid, sid], src_v)
    pltpu.sync_copy(dstp_hbm.at[sid], dst_v)
    plsc.subcore_barrier()
    _sync_agg(hsx_hbm, src_v, dst_v, rows_v, acc, semg, NCH_B)
    plsc.subcore_barrier()
    pltpu.sync_copy(acc.at[pl.ds(r0, rpt)],
                    out_hbm.at[pl.ds(r0, rpt), pl.ds(cid * HH, HH)])


@functools.lru_cache(maxsize=None)
def _sc_agg_big():
    return pl.kernel(
        _sc_agg_big_body,
        out_type=jax.ShapeDtypeStruct((NP, 2 * HH), jnp.float32),
        mesh=plsc.VectorSubcoreMesh(**_MESH),
        scratch_types=[
            pltpu.VMEM((NCH_B, CHUNK_B), jnp.int32),
            pltpu.VMEM((NCH_B, CHUNK_B), jnp.int32),
            pltpu.VMEM((CHUNK_B, HH), jnp.float32),
            pltpu.VMEM_SHARED((NP, HH), jnp.float32),
            pltpu.SemaphoreType.DMA,
        ],
        compiler_params=pltpu.CompilerParams(use_tc_tiling_on_sc=False),
    )


def _tc1_body(degp, x, w1, sp, dis_o, hs1_o, src2_o):
    d = degp[...]
    deg = d[0][:N, 0:1] + d[1][:N, 0:1] + 1.0
    dis = 1.0 / jnp.sqrt(deg)
    dis_o[...] = dis
    h = jnp.dot(x[...], w1[...], preferred_element_type=jnp.float32)
    hs1_o[...] = h * dis
    srcm = sp[...]
    src2_o[0] = srcm * 2
    src2_o[1] = srcm * 2 + 1


_tc1 = pl.pallas_call(
    _tc1_body,
    out_shape=[
        jax.ShapeDtypeStruct((N, 1), jnp.float32),
        jax.ShapeDtypeStruct((N, H), jnp.float32),
        jax.ShapeDtypeStruct((NC, EP_B // 128, 128), jnp.int32),
    ],
)


def _tc_mid_body(agg, hs, dis, b, g, bt, wn, out, *, pad16):
    t = (agg[:N] + hs[...]) * dis[...] + b[...][None, :]
    mu = jnp.mean(t, axis=0, keepdims=True)
    tc = t - mu
    var = jnp.mean(tc * tc, axis=0, keepdims=True)
    y = g[...][None, :] * tc / jnp.sqrt(var + 1e-5) + bt[...][None, :]
    z = jnp.where(y >= 0, y, 0.01 * y)
    hn = jnp.dot(z, wn[...], preferred_element_type=jnp.float32) * dis[...]
    if pad16:
        col = lax.broadcasted_iota(jnp.int32, (1, 16), 1)
        out[:N] = jnp.where(col == 0, hn, 0.0)
        out[N:] = jnp.zeros((NP - N, 16), jnp.float32)
    else:
        out[...] = hn


_tc_mid128 = pl.pallas_call(
    functools.partial(_tc_mid_body, pad16=False),
    out_shape=jax.ShapeDtypeStruct((N, H), jnp.float32),
)

_tc_mid16 = pl.pallas_call(
    functools.partial(_tc_mid_body, pad16=True),
    out_shape=jax.ShapeDtypeStruct((NP, 16), jnp.float32),
)


def _tc3_body(aggp, hs3, dis, b3, out):
    a = aggp[...]
    s = a[0][:N, 0:1] + a[1][:N, 0:1] + hs3[:N, 0:1]
    out[...] = s * dis[...] + b3[...]


_tc3 = pl.pallas_call(
    _tc3_body,
    out_shape=jax.ShapeDtypeStruct((N, 1), jnp.float32),
)


def _pad_edges(v, per_real, per_pad, pad_val, shape):
    r = v.reshape(-1, per_real)
    p = jnp.full((r.shape[0], per_pad - per_real), pad_val, jnp.int32)
    return jnp.concatenate([r, p], axis=1).reshape(shape)


def kernel(x, edge_index, W1, b1, g1, bt1, W2, b2, g2, bt2, W3, b3):
    src = edge_index[0]
    dst = edge_index[1]

    padn = EP_A - E
    src_a = jnp.concatenate([src, jnp.zeros((padn,), jnp.int32)])
    dst_a = jnp.concatenate([dst, jnp.full((padn,), N, jnp.int32)])
    src_r16 = src_a.reshape(NC, NS, NCH_A, CHUNK_A)
    dst_r16 = dst_a.reshape(NC, NS, NCH_A, CHUNK_A)

    src_bp = src.reshape(EP_B // 128, 128)
    dst_bp = dst.reshape(NS, NCH_B, CHUNK_B)

    ones16 = jnp.ones((NP, 16), jnp.float32)
    zeros16 = jnp.zeros((NP, 16), jnp.float32)
    zeros64 = jnp.zeros((NP, HH), jnp.float32)

    degp = _sc_agg16()(ones16, dst_r16, dst_r16, zeros16)

    dis, hs1, src2m = _tc1(degp, x, W1, src_bp)
    src2 = src2m.reshape(NC, NS, NCH_B, CHUNK_B)

    agg1 = _sc_agg_big()(hs1.reshape(2 * N, HH), src2, dst_bp, zeros64)
    hs2 = _tc_mid128(agg1, hs1, dis, b1, g1, bt1, W2)

    agg2 = _sc_agg_big()(hs2.reshape(2 * N, HH), src2, dst_bp, zeros64)
    hs3p = _tc_mid16(agg2, hs2, dis, b2, g2, bt2, W3)

    agg3 = _sc_agg16()(hs3p, src_r16, dst_r16, zeros16)
    return _tc3(agg3, hs3p, dis, b3)


# speedup vs baseline: 1.6418x; 1.0192x over previous
import functools

import jax
import jax.numpy as jnp
from jax import lax
from jax.experimental import pallas as pl
from jax.experimental.pallas import tpu as pltpu
from jax.experimental.pallas import tpu_sc as plsc

N = 10000
E = 320000
D = 128
H = 128
NC = 2
NS = 16
NP = 10240
HH = 64

CHUNK_A = 512
CHUNK_B = 80

ET_A = 10240
NCH_A = ET_A // CHUNK_A
EP_A = ET_A * NC * NS

ET_B = E // NS
NCH_B = ET_B // CHUNK_B
EP_B = E

_MESH = dict(core_axis_name="c", subcore_axis_name="s", num_cores=NC,
             num_subcores=NS)


def _sync_agg(tbl_hbm, src_v, dst_v, rows_v, acc, sem, nch):
    def step(j, carry):
        pltpu.async_copy(tbl_hbm.at[src_v.at[j]], rows_v, sem).wait()
        pltpu.sync_copy(rows_v, acc.at[dst_v.at[j]], add=True)
        return carry

    lax.fori_loop(0, nch, step, 0, unroll=False)


def _sc_agg16_body(tbl_hbm, srcr_hbm, dstr_hbm, zer_hbm, out_hbm,
                   src_v, dst_v, rows_v, acc, semg):
    cid = lax.axis_index("c")
    sid = lax.axis_index("s")
    rpt = NP // NS
    r0 = sid * rpt
    pltpu.sync_copy(zer_hbm.at[pl.ds(r0, rpt)], acc.at[pl.ds(r0, rpt)])
    pltpu.sync_copy(srcr_hbm.at[cid, sid], src_v)
    pltpu.sync_copy(dstr_hbm.at[cid, sid], dst_v)
    plsc.subcore_barrier()
    _sync_agg(tbl_hbm, src_v, dst_v, rows_v, acc, semg, NCH_A)
    plsc.subcore_barrier()
    pltpu.sync_copy(acc.at[pl.ds(r0, rpt)], out_hbm.at[cid, pl.ds(r0, rpt)])


@functools.lru_cache(maxsize=None)
def _sc_agg16():
    return pl.kernel(
        _sc_agg16_body,
        out_type=jax.ShapeDtypeStruct((NC, NP, 16), jnp.float32),
        mesh=plsc.VectorSubcoreMesh(**_MESH),
        scratch_types=[
            pltpu.VMEM((NCH_A, CHUNK_A), jnp.int32),
            pltpu.VMEM((NCH_A, CHUNK_A), jnp.int32),
            pltpu.VMEM((CHUNK_A, 16), jnp.float32),
            pltpu.VMEM_SHARED((NP, 16), jnp.float32),
            pltpu.SemaphoreType.DMA,
        ],
        compiler_params=pltpu.CompilerParams(use_tc_tiling_on_sc=False),
    )


def _sc_agg_big_body(hsx_hbm, src2_hbm, dstp_hbm, zer_hbm, out_hbm,
                     src_v, dst_v, rows_v, acc, semg):
    cid = lax.axis_index("c")
    sid = lax.axis_index("s")
    rpt = NP // NS
    r0 = sid * rpt
    pltpu.sync_copy(zer_hbm.at[pl.ds(r0, rpt)], acc.at[pl.ds(r0, rpt)])
    pltpu.sync_copy(src2_hbm.at[cid, sid], src_v)
    pltpu.sync_copy(dstp_hbm.at[sid], dst_v)
    plsc.subcore_barrier()
    _sync_agg(hsx_hbm, src_v, dst_v, rows_v, acc, semg, NCH_B)
    plsc.subcore_barrier()
    pltpu.sync_copy(acc.at[pl.ds(r0, rpt)],
                    out_hbm.at[pl.ds(r0, rpt), pl.ds(cid * HH, HH)])


@functools.lru_cache(maxsize=None)
def _sc_agg_big():
    return pl.kernel(
        _sc_agg_big_body,
        out_type=jax.ShapeDtypeStruct((NP, 2 * HH), jnp.float32),
        mesh=plsc.VectorSubcoreMesh(**_MESH),
        scratch_types=[
            pltpu.VMEM((NCH_B, CHUNK_B), jnp.int32),
            pltpu.VMEM((NCH_B, CHUNK_B), jnp.int32),
            pltpu.VMEM((CHUNK_B, HH), jnp.float32),
            pltpu.VMEM_SHARED((NP, HH), jnp.float32),
            pltpu.SemaphoreType.DMA,
        ],
        compiler_params=pltpu.CompilerParams(use_tc_tiling_on_sc=False),
    )


def _tc1_body(degp, x, w1, sp, dis_o, hs1_o, src2_o):
    d = degp[...]
    deg = d[0][:N, 0:1] + d[1][:N, 0:1] + 1.0
    dis = 1.0 / jnp.sqrt(deg)
    dis_o[...] = dis
    h = jnp.dot(x[...], w1[...], preferred_element_type=jnp.float32)
    hs1_o[...] = h * dis
    srcm = sp[...]
    src2_o[0] = srcm * 2
    src2_o[1] = srcm * 2 + 1


_tc1 = pl.pallas_call(
    _tc1_body,
    out_shape=[
        jax.ShapeDtypeStruct((N, 1), jnp.float32),
        jax.ShapeDtypeStruct((N, H), jnp.float32),
        jax.ShapeDtypeStruct((NC, EP_B // 128, 128), jnp.int32),
    ],
)


def _tc_mid_body(agg, hs, dis, b, g, bt, wn, out, *, pad16):
    t = (agg[:N] + hs[...]) * dis[...] + b[...][None, :]
    mu = jnp.mean(t, axis=0, keepdims=True)
    tc = t - mu
    var = jnp.mean(tc * tc, axis=0, keepdims=True)
    y = g[...][None, :] * tc / jnp.sqrt(var + 1e-5) + bt[...][None, :]
    z = jnp.where(y >= 0, y, 0.01 * y)
    hn = jnp.dot(z, wn[...], preferred_element_type=jnp.float32) * dis[...]
    if pad16:
        col = lax.broadcasted_iota(jnp.int32, (1, 16), 1)
        out[:N] = jnp.where(col == 0, hn, 0.0)
        out[N:] = jnp.zeros((NP - N, 16), jnp.float32)
    else:
        out[...] = hn


_tc_mid128 = pl.pallas_call(
    functools.partial(_tc_mid_body, pad16=False),
    out_shape=jax.ShapeDtypeStruct((N, H), jnp.float32),
)

_tc_mid16 = pl.pallas_call(
    functools.partial(_tc_mid_body, pad16=True),
    out_shape=jax.ShapeDtypeStruct((NP, 16), jnp.float32),
)


def _tc3_body(aggp, hs3, dis, b3, out):
    a = aggp[...]
    s = a[0][:N, 0:1] + a[1][:N, 0:1] + hs3[:N, 0:1]
    out[...] = s * dis[...] + b3[...]


_tc3 = pl.pallas_call(
    _tc3_body,
    out_shape=jax.ShapeDtypeStruct((N, 1), jnp.float32),
)


def _pad_edges(v, per_real, per_pad, pad_val, shape):
    r = v.reshape(-1, per_real)
    p = jnp.full((r.shape[0], per_pad - per_real), pad_val, jnp.int32)
    return jnp.concatenate([r, p], axis=1).reshape(shape)


def kernel(x, edge_index, W1, b1, g1, bt1, W2, b2, g2, bt2, W3, b3):
    src = edge_index[0]
    dst = edge_index[1]

    padn = EP_A - E
    src_a = jnp.concatenate([src, jnp.zeros((padn,), jnp.int32)])
    dst_a = jnp.concatenate([dst, jnp.full((padn,), N, jnp.int32)])
    src_r16 = src_a.reshape(NC, NS, NCH_A, CHUNK_A)
    dst_r16 = dst_a.reshape(NC, NS, NCH_A, CHUNK_A)

    src_bp = src.reshape(EP_B // 128, 128)
    dst_bp = dst.reshape(NS, NCH_B, CHUNK_B)

    ones16 = jnp.ones((NP, 16), jnp.float32)
    zeros16 = jnp.zeros((NP, 16), jnp.float32)
    zeros64 = jnp.zeros((NP, HH), jnp.float32)

    degp = _sc_agg16()(ones16, dst_r16, dst_r16, zeros16)

    dis, hs1, src2m = _tc1(degp, x, W1, src_bp)
    src2 = src2m.reshape(NC, NS, NCH_B, CHUNK_B)

    agg1 = _sc_agg_big()(hs1.reshape(2 * N, HH), src2, dst_bp, zeros64)
    hs2 = _tc_mid128(agg1, hs1, dis, b1, g1, bt1, W2)

    agg2 = _sc_agg_big()(hs2.reshape(2 * N, HH), src2, dst_bp, zeros64)
    hs3p = _tc_mid16(agg2, hs2, dis, b2, g2, bt2, W3)

    agg3 = _sc_agg16()(hs3p, src_r16, dst_r16, zeros16)
    return _tc3(agg3, hs3p, dis, b3)


# speedup vs baseline: 1.6604x; 1.0113x over previous
import functools

import jax
import jax.numpy as jnp
from jax import lax
from jax.experimental import pallas as pl
from jax.experimental.pallas import tpu as pltpu
from jax.experimental.pallas import tpu_sc as plsc

N = 10000
E = 320000
D = 128
H = 128
NC = 2
NS = 16
NP = 10240
HH = 64

CHUNK_A = 1024
CHUNK_B = 80

ET_A = 10240
NCH_A = ET_A // CHUNK_A
EP_A = ET_A * NC * NS

ET_B = E // NS
NCH_B = ET_B // CHUNK_B
EP_B = E

_MESH = dict(core_axis_name="c", subcore_axis_name="s", num_cores=NC,
             num_subcores=NS)


def _sync_agg(tbl_hbm, src_v, dst_v, rows_v, acc, sem, nch):
    def step(j, carry):
        pltpu.async_copy(tbl_hbm.at[src_v.at[j]], rows_v, sem).wait()
        pltpu.sync_copy(rows_v, acc.at[dst_v.at[j]], add=True)
        return carry

    lax.fori_loop(0, nch, step, 0, unroll=False)


def _sc_agg16_body(tbl_hbm, srcr_hbm, dstr_hbm, zer_hbm, out_hbm,
                   src_v, dst_v, rows_v, acc, semg):
    cid = lax.axis_index("c")
    sid = lax.axis_index("s")
    rpt = NP // NS
    r0 = sid * rpt
    pltpu.sync_copy(zer_hbm.at[pl.ds(r0, rpt)], acc.at[pl.ds(r0, rpt)])
    pltpu.sync_copy(srcr_hbm.at[cid, sid], src_v)
    pltpu.sync_copy(dstr_hbm.at[cid, sid], dst_v)
    plsc.subcore_barrier()
    _sync_agg(tbl_hbm, src_v, dst_v, rows_v, acc, semg, NCH_A)
    plsc.subcore_barrier()
    pltpu.sync_copy(acc.at[pl.ds(r0, rpt)], out_hbm.at[cid, pl.ds(r0, rpt)])


@functools.lru_cache(maxsize=None)
def _sc_agg16():
    return pl.kernel(
        _sc_agg16_body,
        out_type=jax.ShapeDtypeStruct((NC, NP, 16), jnp.float32),
        mesh=plsc.VectorSubcoreMesh(**_MESH),
        scratch_types=[
            pltpu.VMEM((NCH_A, CHUNK_A), jnp.int32),
            pltpu.VMEM((NCH_A, CHUNK_A), jnp.int32),
            pltpu.VMEM((CHUNK_A, 16), jnp.float32),
            pltpu.VMEM_SHARED((NP, 16), jnp.float32),
            pltpu.SemaphoreType.DMA,
        ],
        compiler_params=pltpu.CompilerParams(use_tc_tiling_on_sc=False),
    )


def _sc_agg_big_body(hsx_hbm, src2_hbm, dstp_hbm, zer_hbm, out_hbm,
                     src_v, dst_v, rows_v, acc, semg):
    cid = lax.axis_index("c")
    sid = lax.axis_index("s")
    rpt = NP // NS
    r0 = sid * rpt
    pltpu.sync_copy(zer_hbm.at[pl.ds(r0, rpt)], acc.at[pl.ds(r0, rpt)])
    pltpu.sync_copy(src2_hbm.at[cid, sid], src_v)
    pltpu.sync_copy(dstp_hbm.at[sid], dst_v)
    plsc.subcore_barrier()
    _sync_agg(hsx_hbm, src_v, dst_v, rows_v, acc, semg, NCH_B)
    plsc.subcore_barrier()
    pltpu.sync_copy(acc.at[pl.ds(r0, rpt)],
                    out_hbm.at[pl.ds(r0, rpt), pl.ds(cid * HH, HH)])


@functools.lru_cache(maxsize=None)
def _sc_agg_big():
    return pl.kernel(
        _sc_agg_big_body,
        out_type=jax.ShapeDtypeStruct((NP, 2 * HH), jnp.float32),
        mesh=plsc.VectorSubcoreMesh(**_MESH),
        scratch_types=[
            pltpu.VMEM((NCH_B, CHUNK_B), jnp.int32),
            pltpu.VMEM((NCH_B, CHUNK_B), jnp.int32),
            pltpu.VMEM((CHUNK_B, HH), jnp.float32),
            pltpu.VMEM_SHARED((NP, HH), jnp.float32),
            pltpu.SemaphoreType.DMA,
        ],
        compiler_params=pltpu.CompilerParams(use_tc_tiling_on_sc=False),
    )


def _tc1_body(degp, x, w1, sp, dis_o, hs1_o, src2_o):
    d = degp[...]
    deg = d[0][:N, 0:1] + d[1][:N, 0:1] + 1.0
    dis = 1.0 / jnp.sqrt(deg)
    dis_o[...] = dis
    h = jnp.dot(x[...], w1[...], preferred_element_type=jnp.float32)
    hs1_o[...] = h * dis
    srcm = sp[...]
    src2_o[0] = srcm * 2
    src2_o[1] = srcm * 2 + 1


_tc1 = pl.pallas_call(
    _tc1_body,
    out_shape=[
        jax.ShapeDtypeStruct((N, 1), jnp.float32),
        jax.ShapeDtypeStruct((N, H), jnp.float32),
        jax.ShapeDtypeStruct((NC, EP_B // 128, 128), jnp.int32),
    ],
)


def _tc_mid_body(agg, hs, dis, b, g, bt, wn, out, *, pad16):
    t = (agg[:N] + hs[...]) * dis[...] + b[...][None, :]
    mu = jnp.mean(t, axis=0, keepdims=True)
    tc = t - mu
    var = jnp.mean(tc * tc, axis=0, keepdims=True)
    y = g[...][None, :] * tc / jnp.sqrt(var + 1e-5) + bt[...][None, :]
    z = jnp.where(y >= 0, y, 0.01 * y)
    hn = jnp.dot(z, wn[...], preferred_element_type=jnp.float32) * dis[...]
    if pad16:
        col = lax.broadcasted_iota(jnp.int32, (1, 16), 1)
        out[:N] = jnp.where(col == 0, hn, 0.0)
        out[N:] = jnp.zeros((NP - N, 16), jnp.float32)
    else:
        out[...] = hn


_tc_mid128 = pl.pallas_call(
    functools.partial(_tc_mid_body, pad16=False),
    out_shape=jax.ShapeDtypeStruct((N, H), jnp.float32),
)

_tc_mid16 = pl.pallas_call(
    functools.partial(_tc_mid_body, pad16=True),
    out_shape=jax.ShapeDtypeStruct((NP, 16), jnp.float32),
)


def _tc3_body(aggp, hs3, dis, b3, out):
    a = aggp[...]
    s = a[0][:N, 0:1] + a[1][:N, 0:1] + hs3[:N, 0:1]
    out[...] = s * dis[...] + b3[...]


_tc3 = pl.pallas_call(
    _tc3_body,
    out_shape=jax.ShapeDtypeStruct((N, 1), jnp.float32),
)


def _pad_edges(v, per_real, per_pad, pad_val, shape):
    r = v.reshape(-1, per_real)
    p = jnp.full((r.shape[0], per_pad - per_real), pad_val, jnp.int32)
    return jnp.concatenate([r, p], axis=1).reshape(shape)


def kernel(x, edge_index, W1, b1, g1, bt1, W2, b2, g2, bt2, W3, b3):
    src = edge_index[0]
    dst = edge_index[1]

    padn = EP_A - E
    src_a = jnp.concatenate([src, jnp.zeros((padn,), jnp.int32)])
    dst_a = jnp.concatenate([dst, jnp.full((padn,), N, jnp.int32)])
    src_r16 = src_a.reshape(NC, NS, NCH_A, CHUNK_A)
    dst_r16 = dst_a.reshape(NC, NS, NCH_A, CHUNK_A)

    src_bp = src.reshape(EP_B // 128, 128)
    dst_bp = dst.reshape(NS, NCH_B, CHUNK_B)

    ones16 = jnp.ones((NP, 16), jnp.float32)
    zeros16 = jnp.zeros((NP, 16), jnp.float32)
    zeros64 = jnp.zeros((NP, HH), jnp.float32)

    degp = _sc_agg16()(ones16, dst_r16, dst_r16, zeros16)

    dis, hs1, src2m = _tc1(degp, x, W1, src_bp)
    src2 = src2m.reshape(NC, NS, NCH_B, CHUNK_B)

    agg1 = _sc_agg_big()(hs1.reshape(2 * N, HH), src2, dst_bp, zeros64)
    hs2 = _tc_mid128(agg1, hs1, dis, b1, g1, bt1, W2)

    agg2 = _sc_agg_big()(hs2.reshape(2 * N, HH), src2, dst_bp, zeros64)
    hs3p = _tc_mid16(agg2, hs2, dis, b2, g2, bt2, W3)

    agg3 = _sc_agg16()(hs3p, src_r16, dst_r16, zeros16)
    return _tc3(agg3, hs3p, dis, b3)
